# Initial kernel scaffold; baseline (speedup 1.0000x reference)
import jax
import jax.numpy as jnp
from jax.experimental import pallas as pl


def kernel(data, edge_index, W1, b1, W2, b2):
    raise NotImplementedError("write your pallas kernel here")


# speedup vs baseline: 2.6842x; 2.6842x over previous
import jax
import jax.numpy as jnp
from jax import lax
from jax.experimental import pallas as pl
from jax.experimental.pallas import tpu as pltpu
from jax.experimental.pallas import tpu_sc as plsc

N = 10000
E = 320000
D = 128
NC = 2
NS = 16
NW = NC * NS
CH = 128
NCH = 80
EPT = NCH * CH
EPAD = NW * EPT
NP = 10240
TRASH = N
RPT = NP // NS


def _spmm2_body(x_hbm, srcA_hbm, srcB_hbm, dst_hbm,
                outp_hbm, outq_hbm, outd_hbm,
                sidx0, sidx1, didx0, didx1, rows0, rows1, zrow, ones_v,
                semis0, semid0, semis1, semid1, semr0, semr1,
                acc, dacc):
    cid = lax.axis_index("c")
    sid = lax.axis_index("s")
    wid = cid * NS + sid
    rbase = sid * RPT
    obase = cid * NP + rbase

    def _zero_rows0():
        def _zr(i, _):
            for j in range(D // 16):
                rows0[i, pl.ds(j * 16, 16)] = jnp.zeros((16,), jnp.float32)
            return 0
        lax.fori_loop(0, CH, _zr, 0)

    def _clear_acc():
        for r in range(RPT // CH):
            pltpu.sync_copy(rows0, acc.at[pl.ds(rbase + r * CH, CH)])

    def _z1(k, _):
        zrow[pl.ds(k * 16, 16)] = jnp.zeros((16,), jnp.float32)
        return 0
    lax.fori_loop(0, RPT // 16, _z1, 0)
    for j in range(CH // 16):
        ones_v[pl.ds(j * 16, 16)] = jnp.ones((16,), jnp.float32)

    _zero_rows0()
    _clear_acc()
    pltpu.sync_copy(zrow, dacc.at[pl.ds(rbase, RPT)])
    plsc.subcore_barrier()

    def _phase(sidx_hbm, sbase, dbase, nch, table_hbm, count_degrees):
        slots = ((sidx0, didx0, semis0, semid0, rows0, semr0),
                 (sidx1, didx1, semis1, semid1, rows1, semr1))

        def _idx_load(c, s):
            sb, db, ss, sd = slots[s][:4]
            pltpu.async_copy(sidx_hbm.at[pl.ds((sbase + c) * CH, CH)], sb, ss)
            pltpu.async_copy(dst_hbm.at[pl.ds((dbase + c) * CH, CH)], db, sd)

        def _idx_wait(c, s):
            sb, db, ss, sd = slots[s][:4]
            pltpu.make_async_copy(
                sidx_hbm.at[pl.ds((sbase + c) * CH, CH)], sb, ss).wait()
            pltpu.make_async_copy(
                dst_hbm.at[pl.ds((dbase + c) * CH, CH)], db, sd).wait()

        def _gather(s):
            sb, _, _, _, rw, sr = slots[s]
            pltpu.async_copy(table_hbm.at[sb], rw, sr)

        def _gather_wait(s):
            sb, _, _, _, rw, sr = slots[s]
            pltpu.make_async_copy(table_hbm.at[sb], rw, sr).wait()

        def _scatter(s):
            _, db, _, _, rw, _ = slots[s]
            pltpu.sync_copy(rw, acc.at[db], add=True)
            if count_degrees:
                pltpu.sync_copy(ones_v, dacc.at[db], add=True)

        _idx_load(0, 0)
        _idx_load(1, 1)
        _idx_wait(0, 0)
        _gather(0)

        def _half(c, s):
            _idx_wait(c + 1, 1 - s)
            _gather(1 - s)
            _gather_wait(s)
            _scatter(s)
            _idx_load(c + 2, s)

        def _pair(k, _):
            _half(2 * k, 0)
            _half(2 * k + 1, 1)
            return 0
        lax.fori_loop(0, nch // 2 - 1, _pair, 0)

        _idx_wait(nch - 1, 1)
        _gather(1)
        _gather_wait(0)
        _scatter(0)
        _gather_wait(1)
        _scatter(1)

    _phase(srcA_hbm, wid * NCH, wid * NCH, NCH, x_hbm, True)
    plsc.subcore_barrier()

    pltpu.sync_copy(acc.at[pl.ds(rbase, RPT)], outp_hbm.at[pl.ds(obase, RPT)])
    pltpu.sync_copy(dacc.at[pl.ds(rbase, RPT)], outd_hbm.at[pl.ds(obase, RPT)])
    _zero_rows0()
    _clear_acc()
    plsc.subcore_barrier()

    NCH2 = NW * NCH // NS
    _phase(srcB_hbm, cid * NW * NCH + sid * NCH2, sid * NCH2, NCH2,
           outp_hbm, False)
    plsc.subcore_barrier()

    pltpu.sync_copy(acc.at[pl.ds(rbase, RPT)], outq_hbm.at[pl.ds(obase, RPT)])


_spmm2 = pl.kernel(
    _spmm2_body,
    out_type=(
        jax.ShapeDtypeStruct((NC * NP, D), jnp.float32),
        jax.ShapeDtypeStruct((NC * NP, D), jnp.float32),
        jax.ShapeDtypeStruct((NC * NP,), jnp.float32),
    ),
    mesh=plsc.VectorSubcoreMesh(core_axis_name="c", subcore_axis_name="s",
                                num_cores=NC, num_subcores=NS),
    scratch_types=[
        pltpu.VMEM((CH,), jnp.int32),
        pltpu.VMEM((CH,), jnp.int32),
        pltpu.VMEM((CH,), jnp.int32),
        pltpu.VMEM((CH,), jnp.int32),
        pltpu.VMEM((CH, D), jnp.float32),
        pltpu.VMEM((CH, D), jnp.float32),
        pltpu.VMEM((RPT,), jnp.float32),
        pltpu.VMEM((CH,), jnp.float32),
        pltpu.SemaphoreType.DMA,
        pltpu.SemaphoreType.DMA,
        pltpu.SemaphoreType.DMA,
        pltpu.SemaphoreType.DMA,
        pltpu.SemaphoreType.DMA,
        pltpu.SemaphoreType.DMA,
        pltpu.VMEM_SHARED((NP, D), jnp.float32),
        pltpu.VMEM_SHARED((NP,), jnp.float32),
    ],
)


_BM = 1024


def _final_body(q0_ref, q1_ref, d_ref, w1_ref, w2_ref, b1_ref, b2_ref, o_ref):
    hi = jax.lax.Precision.HIGHEST
    w12 = jnp.dot(w1_ref[...], w2_ref[...], precision=hi,
                  preferred_element_type=jnp.float32)
    bw = jnp.dot(b1_ref[...], w2_ref[...], precision=hi,
                 preferred_element_type=jnp.float32)
    z = q0_ref[...] + q1_ref[...]
    d = d_ref[0] + d_ref[1]
    o_ref[...] = (jnp.dot(z, w12, precision=hi,
                          preferred_element_type=jnp.float32)
                  + d[:, None] * bw + b2_ref[...])


def _final(q, deg, W1, W2, b1r, b2r):
    return pl.pallas_call(
        _final_body,
        grid=(NP // _BM,),
        in_specs=[pl.BlockSpec((_BM, D), lambda i: (i, 0)),
                  pl.BlockSpec((_BM, D), lambda i: (NP // _BM + i, 0)),
                  pl.BlockSpec((NC, _BM), lambda i: (0, i)),
                  pl.BlockSpec((D, D), lambda i: (0, 0)),
                  pl.BlockSpec((D, D), lambda i: (0, 0)),
                  pl.BlockSpec((1, D), lambda i: (0, 0)),
                  pl.BlockSpec((1, D), lambda i: (0, 0))],
        out_specs=pl.BlockSpec((_BM, D), lambda i: (i, 0)),
        out_shape=jax.ShapeDtypeStruct((NP, D), jnp.float32),
    )(q, q, deg, W1, W2, b1r, b2r)


def kernel(data, edge_index, W1, b1, W2, b2):
    src = edge_index[0].astype(jnp.int32)
    dst = edge_index[1].astype(jnp.int32)
    pad = EPAD - E
    src = jnp.concatenate([src, jnp.zeros((pad,), jnp.int32)])
    dst = jnp.concatenate([dst, jnp.full((pad,), TRASH, jnp.int32)])
    srcB = jnp.concatenate([src, src + NP])
    b1r = b1.reshape(1, D)
    b2r = b2.reshape(1, D)

    _p, q, deg = _spmm2(data, src, srcB, dst)
    out = _final(q, deg.reshape(NC, NP), W1, W2, b1r, b2r)
    return out[:N]


# speedup vs baseline: 2.6853x; 1.0004x over previous
import jax
import jax.numpy as jnp
from jax import lax
from jax.experimental import pallas as pl
from jax.experimental.pallas import tpu as pltpu
from jax.experimental.pallas import tpu_sc as plsc

N = 10000
E = 320000
D = 128
NC = 2
NS = 16
NW = NC * NS
CH = 128
NCH = 80
EPT = NCH * CH
EPAD = NW * EPT
NP = 10240
TRASH = N
RPT = NP // NS


def _spmm2_body(x_hbm, srcA_hbm, srcB_hbm, dst_hbm,
                outp_hbm, outq_hbm, outd_hbm,
                sidx0, sidx1, didx0, didx1, rows0, rows1, zrow, ones_v,
                semis0, semid0, semis1, semid1, semr0, semr1,
                acc, dacc):
    cid = lax.axis_index("c")
    sid = lax.axis_index("s")
    wid = cid * NS + sid
    rbase = sid * RPT
    obase = cid * NP + rbase

    def _zero_rows0():
        def _zr(i, _):
            for j in range(D // 16):
                rows0[i, pl.ds(j * 16, 16)] = jnp.zeros((16,), jnp.float32)
            return 0
        lax.fori_loop(0, CH, _zr, 0)

    def _clear_acc():
        for r in range(RPT // CH):
            pltpu.sync_copy(rows0, acc.at[pl.ds(rbase + r * CH, CH)])

    def _z1(k, _):
        zrow[pl.ds(k * 16, 16)] = jnp.zeros((16,), jnp.float32)
        return 0
    lax.fori_loop(0, RPT // 16, _z1, 0)
    for j in range(CH // 16):
        ones_v[pl.ds(j * 16, 16)] = jnp.ones((16,), jnp.float32)

    _zero_rows0()
    _clear_acc()
    pltpu.sync_copy(zrow, dacc.at[pl.ds(rbase, RPT)])
    plsc.subcore_barrier()

    def _phase(sidx_hbm, sbase, dbase, nch, table_hbm, count_degrees):
        slots = ((sidx0, didx0, semis0, semid0, rows0, semr0),
                 (sidx1, didx1, semis1, semid1, rows1, semr1))

        def _idx_load(c, s):
            sb, db, ss, sd = slots[s][:4]
            pltpu.async_copy(sidx_hbm.at[pl.ds((sbase + c) * CH, CH)], sb, ss)
            pltpu.async_copy(dst_hbm.at[pl.ds((dbase + c) * CH, CH)], db, sd)

        def _idx_wait(c, s):
            sb, db, ss, sd = slots[s][:4]
            pltpu.make_async_copy(
                sidx_hbm.at[pl.ds((sbase + c) * CH, CH)], sb, ss).wait()
            pltpu.make_async_copy(
                dst_hbm.at[pl.ds((dbase + c) * CH, CH)], db, sd).wait()

        def _gather(s):
            sb, _, _, _, rw, sr = slots[s]
            pltpu.async_copy(table_hbm.at[sb], rw, sr)

        def _gather_wait(s):
            sb, _, _, _, rw, sr = slots[s]
            pltpu.make_async_copy(table_hbm.at[sb], rw, sr).wait()

        def _scatter(s):
            _, db, _, _, rw, _ = slots[s]
            pltpu.sync_copy(rw, acc.at[db], add=True)
            if count_degrees:
                pltpu.sync_copy(ones_v, dacc.at[db], add=True)

        _idx_load(0, 0)
        _idx_load(1, 1)
        _idx_wait(0, 0)
        _gather(0)

        def _half(c, s):
            _idx_wait(c + 1, 1 - s)
            _gather(1 - s)
            _gather_wait(s)
            _scatter(s)
            _idx_load(c + 2, s)

        def _pair(k, _):
            _half(2 * k, 0)
            _half(2 * k + 1, 1)
            return 0
        lax.fori_loop(0, nch // 2 - 1, _pair, 0)

        _idx_wait(nch - 1, 1)
        _gather(1)
        _gather_wait(0)
        _scatter(0)
        _gather_wait(1)
        _scatter(1)

    _phase(srcA_hbm, wid * NCH, wid * NCH, NCH, x_hbm, True)
    plsc.subcore_barrier()

    pltpu.sync_copy(acc.at[pl.ds(rbase, RPT)], outp_hbm.at[pl.ds(obase, RPT)])
    pltpu.sync_copy(dacc.at[pl.ds(rbase, RPT)], outd_hbm.at[pl.ds(obase, RPT)])
    _zero_rows0()
    _clear_acc()
    plsc.subcore_barrier()

    NCH2 = NW * NCH // NS
    _phase(srcB_hbm, cid * NW * NCH + sid * NCH2, sid * NCH2, NCH2,
           outp_hbm, False)
    plsc.subcore_barrier()

    pltpu.sync_copy(acc.at[pl.ds(rbase, RPT)], outq_hbm.at[pl.ds(obase, RPT)])


_spmm2 = pl.kernel(
    _spmm2_body,
    out_type=(
        jax.ShapeDtypeStruct((NC * NP, D), jnp.float32),
        jax.ShapeDtypeStruct((NC * NP, D), jnp.float32),
        jax.ShapeDtypeStruct((NC * NP,), jnp.float32),
    ),
    mesh=plsc.VectorSubcoreMesh(core_axis_name="c", subcore_axis_name="s",
                                num_cores=NC, num_subcores=NS),
    scratch_types=[
        pltpu.VMEM((CH,), jnp.int32),
        pltpu.VMEM((CH,), jnp.int32),
        pltpu.VMEM((CH,), jnp.int32),
        pltpu.VMEM((CH,), jnp.int32),
        pltpu.VMEM((CH, D), jnp.float32),
        pltpu.VMEM((CH, D), jnp.float32),
        pltpu.VMEM((RPT,), jnp.float32),
        pltpu.VMEM((CH,), jnp.float32),
        pltpu.SemaphoreType.DMA,
        pltpu.SemaphoreType.DMA,
        pltpu.SemaphoreType.DMA,
        pltpu.SemaphoreType.DMA,
        pltpu.SemaphoreType.DMA,
        pltpu.SemaphoreType.DMA,
        pltpu.VMEM_SHARED((NP, D), jnp.float32),
        pltpu.VMEM_SHARED((NP,), jnp.float32),
    ],
)


_BM = 1024


def _final_body(q0_ref, q1_ref, d_ref, w1_ref, w2_ref, b1_ref, b2_ref, o_ref):
    hi = jax.lax.Precision.HIGHEST
    w12 = jnp.dot(w1_ref[...], w2_ref[...], precision=hi,
                  preferred_element_type=jnp.float32)
    bw = jnp.dot(b1_ref[...], w2_ref[...], precision=hi,
                 preferred_element_type=jnp.float32)
    z = q0_ref[...] + q1_ref[...]
    d = d_ref[0] + d_ref[1]
    o_ref[...] = (jnp.dot(z, w12, precision=hi,
                          preferred_element_type=jnp.float32)
                  + d[:, None] * bw + b2_ref[...])


def _final(q, deg, W1, W2, b1r, b2r):
    return pl.pallas_call(
        _final_body,
        grid=(NP // _BM,),
        in_specs=[pl.BlockSpec((_BM, D), lambda i: (i, 0)),
                  pl.BlockSpec((_BM, D), lambda i: (NP // _BM + i, 0)),
                  pl.BlockSpec((NC, _BM), lambda i: (0, i)),
                  pl.BlockSpec((D, D), lambda i: (0, 0)),
                  pl.BlockSpec((D, D), lambda i: (0, 0)),
                  pl.BlockSpec((1, D), lambda i: (0, 0)),
                  pl.BlockSpec((1, D), lambda i: (0, 0))],
        out_specs=pl.BlockSpec((_BM, D), lambda i: (i, 0)),
        out_shape=jax.ShapeDtypeStruct((NP, D), jnp.float32),
    )(q, q, deg, W1, W2, b1r, b2r)


def kernel(data, edge_index, W1, b1, W2, b2):
    src = edge_index[0].astype(jnp.int32)
    dst = edge_index[1].astype(jnp.int32)
    pad = EPAD - E
    src = jnp.concatenate([src, jnp.zeros((pad,), jnp.int32)])
    dst = jnp.concatenate(
        [dst, TRASH + jnp.arange(pad, dtype=jnp.int32) % (NP - N)])
    srcB = jnp.concatenate([src, src + NP])
    b1r = b1.reshape(1, D)
    b2r = b2.reshape(1, D)

    _p, q, deg = _spmm2(data, src, srcB, dst)
    out = _final(q, deg.reshape(NC, NP), W1, W2, b1r, b2r)
    return out[:N]
